# Initial kernel scaffold; baseline (speedup 1.0000x reference)
import jax
import jax.numpy as jnp
from jax.experimental import pallas as pl


def kernel(inputs):
    raise NotImplementedError("write your pallas kernel here")


# speedup vs baseline: 34.0808x; 34.0808x over previous
import jax
import jax.numpy as jnp
from jax.experimental import pallas as pl
from jax.experimental.pallas import tpu as pltpu

_B = 64
_N = 1_000_000
_K = 2048
_CHUNK = 1024
_WOUT = 65536


def _cumsum_lanes(x):
    n = x.shape[1]
    zeros_cache = {}
    s = 1
    while s < n:
        if s not in zeros_cache:
            zeros_cache[s] = jnp.zeros((x.shape[0], s), x.dtype)
        x = x + jnp.concatenate([zeros_cache[s], x[:, : n - s]], axis=1)
        s *= 2
    return x


def _prefix_scan_kernel(x_ref, sv_ref, idx_ref, done_ref):
    ics = _cumsum_lanes(x_ref[...])
    cnt = jnp.sum((ics < sv_ref[...]).astype(jnp.int32), axis=1, keepdims=True)
    idx_ref[...] = cnt
    done_ref[...] = (cnt < _K).astype(jnp.int32)


def _full_scan_kernel(x_ref, sv_ref, idx_ref, carry_ref, acc_ref, done_ref):
    k = pl.program_id(0)

    @pl.when(k == 0)
    def _init():
        carry_ref[...] = jnp.zeros_like(carry_ref)
        acc_ref[...] = jnp.zeros_like(acc_ref)
        done_ref[...] = jnp.zeros_like(done_ref)

    col = k * _CHUNK + jax.lax.broadcasted_iota(jnp.int32, (_B, _CHUNK), 1)
    valid = col < _N
    x = jnp.where(valid, x_ref[...], 0.0)
    ics = carry_ref[...] + _cumsum_lanes(x)
    lt = jnp.logical_and(ics < sv_ref[...], valid)
    cnt = jnp.sum(lt.astype(jnp.int32), axis=1, keepdims=True)
    nvalid = jnp.sum(valid.astype(jnp.int32), axis=1, keepdims=True)
    done = done_ref[...]
    acc_ref[...] = acc_ref[...] + jnp.where(done > 0, 0, cnt)
    done_ref[...] = jnp.maximum(done, (cnt < nvalid).astype(jnp.int32))
    carry_ref[...] = ics[:, _CHUNK - 1:_CHUNK]

    @pl.when(k == pl.num_programs(0) - 1)
    def _emit():
        idx_ref[...] = acc_ref[...]


def _onehot_kernel(idx_ref, o_ref):
    j = pl.program_id(0)
    col = j * _WOUT + jax.lax.broadcasted_iota(jnp.int32, o_ref.shape, 1)
    o_ref[...] = (col == idx_ref[...]).astype(jnp.float32)


def _full_scan(inputs, sv):
    return pl.pallas_call(
        _full_scan_kernel,
        grid=(pl.cdiv(_N, _CHUNK),),
        in_specs=[
            pl.BlockSpec((_B, _CHUNK), lambda k: (0, k)),
            pl.BlockSpec((_B, 1), lambda k: (0, 0)),
        ],
        out_specs=pl.BlockSpec((_B, 1), lambda k: (0, 0)),
        out_shape=jax.ShapeDtypeStruct((_B, 1), jnp.int32),
        scratch_shapes=[
            pltpu.VMEM((_B, 1), jnp.float32),
            pltpu.VMEM((_B, 1), jnp.int32),
            pltpu.VMEM((_B, 1), jnp.int32),
        ],
    )(inputs, sv)


def kernel(inputs):
    skey = jax.random.fold_in(jax.random.key(0), 1)
    sv = jax.random.uniform(skey, (_B, 1), dtype=inputs.dtype,
                            minval=0.0, maxval=1.0)

    idx0, done0 = pl.pallas_call(
        _prefix_scan_kernel,
        out_shape=(
            jax.ShapeDtypeStruct((_B, 1), jnp.int32),
            jax.ShapeDtypeStruct((_B, 1), jnp.int32),
        ),
    )(inputs[:, :_K], sv)

    idx = jax.lax.cond(
        jnp.all(done0 > 0),
        lambda: idx0,
        lambda: _full_scan(inputs, sv),
    )

    return pl.pallas_call(
        _onehot_kernel,
        grid=(pl.cdiv(_N, _WOUT),),
        in_specs=[pl.BlockSpec((_B, 1), lambda j: (0, 0))],
        out_specs=pl.BlockSpec((_B, _WOUT), lambda j: (0, j)),
        out_shape=jax.ShapeDtypeStruct((_B, _N), jnp.float32),
    )(idx)
